# baseline (speedup 1.0000x reference)
import jax
import jax.numpy as jnp
from jax.experimental import pallas as pl

L = 24
N = 1024
D_IN = 64
H = 128


def _gru_scan(x_seq, h0, Wih, Whh, bih, bhh):
    def step(h, x):
        gi = Wih @ x + bih
        gh = Whh @ h + bhh
        i_r, i_z, i_n = jnp.split(gi, 3)
        h_r, h_z, h_n = jnp.split(gh, 3)
        r = jax.nn.sigmoid(i_r + h_r)
        z = jax.nn.sigmoid(i_z + h_z)
        n = jnp.tanh(i_n + r * h_n)
        h_new = (1.0 - z) * n + z * h
        return h_new, h_new
    _, hs = jax.lax.scan(step, h0, x_seq)
    return hs


def _layer_kernel(sig_ref, mask_ref, out_ref):
    cur_row = sig_ref[:].reshape(1, N)
    cur_col = cur_row.reshape(N, 1)

    ii = jax.lax.broadcasted_iota(jnp.int32, (N, N), 0)
    jj = jax.lax.broadcasted_iota(jnp.int32, (N, N), 1)
    tie = (cur_row == cur_col) & (jj < ii)
    lt = (cur_row < cur_col) | tie
    gt = (cur_row > cur_col) | tie
    one = jnp.float32(1.0)
    zero = jnp.float32(0.0)
    rank0 = jnp.sum(jnp.where(lt, one, zero), axis=1, keepdims=True)
    inv = jnp.sum(jnp.where(gt, one, zero), axis=1, keepdims=True)

    kk = jnp.asarray(jj, jnp.float32)
    Q = jnp.where(inv == kk, one, zero)
    mask_row = mask_ref[:].reshape(1, N)
    c_mask = jax.lax.dot_general(Q, mask_row, (((1,), (1,)), ((), ())),
                                 preferred_element_type=jnp.float32,
                                 precision=jax.lax.Precision.HIGHEST)

    delta = kk - rank0
    num = jnp.exp(-delta * delta)
    den = jnp.sum(num, axis=1, keepdims=True)
    soft = jax.lax.dot_general(num, c_mask, (((1,), (0,)), ((), ())),
                               preferred_element_type=jnp.float32,
                               precision=jax.lax.Precision.HIGHEST) / den
    hard = jnp.where(soft >= 0.5, one, zero)
    soft_x = (soft - hard) + soft

    out_row = jax.lax.dot_general(soft_x, Q, (((0,), (1,)), ((), ())),
                                  preferred_element_type=jnp.float32,
                                  precision=jax.lax.Precision.HIGHEST)
    out_ref[:] = out_row.reshape(1, 1, N)


def kernel(k_masks, inputs, h0, Wih_f, Whh_f, bih_f, bhh_f, Wih_b, Whh_b,
           bih_b, bhh_b, ln_w, ln_b, W_fc):
    x = inputs[:, 0, :]
    hf = _gru_scan(x, h0[0, 0], Wih_f, Whh_f, bih_f, bhh_f)
    hb = _gru_scan(x[::-1], h0[1, 0], Wih_b, Whh_b, bih_b, bhh_b)[::-1]
    out = jnp.concatenate([hf, hb], axis=1)
    mu = out.mean(axis=-1, keepdims=True)
    var = ((out - mu) ** 2).mean(axis=-1, keepdims=True)
    ln = (out - mu) / jnp.sqrt(var + 1e-5) * ln_w + ln_b
    act = jax.nn.relu(ln)
    logits = jnp.einsum('lnc,lc->ln', W_fc, act)
    sig = jax.nn.sigmoid(logits)

    out3 = pl.pallas_call(
        _layer_kernel,
        grid=(L,),
        in_specs=[
            pl.BlockSpec((1, 1, N), lambda l: (l, 0, 0)),
            pl.BlockSpec((1, 1, N), lambda l: (l, 0, 0)),
        ],
        out_specs=pl.BlockSpec((1, 1, N), lambda l: (l, 0, 0)),
        out_shape=jax.ShapeDtypeStruct((L, 1, N), jnp.float32),
    )(sig.reshape(L, 1, N), k_masks.reshape(L, 1, N))
    return out3.reshape(L, N)


# speedup vs baseline: 1.9652x; 1.9652x over previous
import math

import jax
import jax.numpy as jnp
from jax.experimental import pallas as pl

L = 24
N = 1024
D_IN = 64
H = 128

_BAND = 5
_W = [math.exp(-float(d * d)) for d in range(-_BAND, _BAND + 1)]


def _gru_scan(x_seq, h0, Wih, Whh, bih, bhh):
    def step(h, x):
        gi = Wih @ x + bih
        gh = Whh @ h + bhh
        i_r, i_z, i_n = jnp.split(gi, 3)
        h_r, h_z, h_n = jnp.split(gh, 3)
        r = jax.nn.sigmoid(i_r + h_r)
        z = jax.nn.sigmoid(i_z + h_z)
        n = jnp.tanh(i_n + r * h_n)
        h_new = (1.0 - z) * n + z * h
        return h_new, h_new
    _, hs = jax.lax.scan(step, h0, x_seq)
    return hs


def _shift(row, d):
    if d == 0:
        return row
    zeros = jnp.zeros((1, abs(d)), jnp.float32)
    if d > 0:
        return jnp.concatenate([row[:, d:], zeros], axis=1)
    return jnp.concatenate([zeros, row[:, :N + d]], axis=1)


def _layer_kernel(sig_ref, mask_ref, out_ref):
    cur_row = sig_ref[:].reshape(1, N)
    cur_col = cur_row.reshape(N, 1)

    ii = jax.lax.broadcasted_iota(jnp.int32, (N, N), 1)
    jj = jax.lax.broadcasted_iota(jnp.int32, (N, N), 0)
    tie = (cur_col == cur_row) & (jj < ii)
    lt = (cur_col < cur_row) | tie
    gt = (cur_col > cur_row) | tie
    one = jnp.float32(1.0)
    zero = jnp.float32(0.0)
    rank0 = jnp.sum(jnp.where(lt, one, zero), axis=0, keepdims=True)
    inv = jnp.sum(jnp.where(gt, one, zero), axis=0, keepdims=True)

    kk_col = jnp.asarray(jax.lax.broadcasted_iota(jnp.int32, (N, N), 0),
                         jnp.float32)
    QT = jnp.where(kk_col == inv, one, zero)
    mask_row = mask_ref[:].reshape(1, N)
    lhs = jnp.concatenate([mask_row, rank0], axis=0)
    gathered = jax.lax.dot_general(lhs, QT, (((1,), (0,)), ((), ())),
                                   preferred_element_type=jnp.float32,
                                   precision=jax.lax.Precision.HIGHEST)
    c_row = gathered[0:1, :]
    p_row = gathered[1:2, :]

    pos = jnp.asarray(jax.lax.broadcasted_iota(jnp.int32, (1, N), 1),
                      jnp.float32)
    num = jnp.zeros((1, N), jnp.float32)
    den = jnp.zeros((1, N), jnp.float32)
    for d in range(-_BAND, _BAND + 1):
        w = jnp.float32(_W[d + _BAND])
        num = num + w * _shift(c_row, d)
        shifted_pos = pos + jnp.float32(d)
        valid = (shifted_pos >= zero) & (shifted_pos < jnp.float32(N))
        den = den + jnp.where(valid, w, zero)
    soft = num / den
    hard = jnp.where(soft >= jnp.float32(0.5), one, zero)
    t_row = (soft - hard) + soft

    PT = jnp.where(kk_col == p_row, one, zero)
    out_row = jax.lax.dot_general(t_row, PT, (((1,), (0,)), ((), ())),
                                  preferred_element_type=jnp.float32,
                                  precision=jax.lax.Precision.HIGHEST)
    out_ref[:] = out_row.reshape(1, 1, N)


def kernel(k_masks, inputs, h0, Wih_f, Whh_f, bih_f, bhh_f, Wih_b, Whh_b,
           bih_b, bhh_b, ln_w, ln_b, W_fc):
    x = inputs[:, 0, :]
    hf = _gru_scan(x, h0[0, 0], Wih_f, Whh_f, bih_f, bhh_f)
    hb = _gru_scan(x[::-1], h0[1, 0], Wih_b, Whh_b, bih_b, bhh_b)[::-1]
    out = jnp.concatenate([hf, hb], axis=1)
    mu = out.mean(axis=-1, keepdims=True)
    var = ((out - mu) ** 2).mean(axis=-1, keepdims=True)
    ln = (out - mu) / jnp.sqrt(var + 1e-5) * ln_w + ln_b
    act = jax.nn.relu(ln)
    logits = jnp.einsum('lnc,lc->ln', W_fc, act)
    sig = jax.nn.sigmoid(logits)

    out3 = pl.pallas_call(
        _layer_kernel,
        grid=(L,),
        in_specs=[
            pl.BlockSpec((1, 1, N), lambda l: (l, 0, 0)),
            pl.BlockSpec((1, 1, N), lambda l: (l, 0, 0)),
        ],
        out_specs=pl.BlockSpec((1, 1, N), lambda l: (l, 0, 0)),
        out_shape=jax.ShapeDtypeStruct((L, 1, N), jnp.float32),
    )(sig.reshape(L, 1, N), k_masks.reshape(L, 1, N))
    return out3.reshape(L, N)
